# baseline (speedup 1.0000x reference)
import functools

import jax
import jax.numpy as jnp
import numpy as np
from jax import lax
from jax.experimental import pallas as pl
from jax.experimental.pallas import tpu as pltpu
from jax.experimental.pallas import tpu_sc as plsc

_B = 16384
_L = 200
_H = 64
_BT = _B * _L
_NP = _BT // 2
_PPR = _L // 2
_NW = 32
_PPW = _NP // _NW
_PCHUNK = 200
_NIDX = 100
_NCH = _PPW // _PCHUNK
_SROWS = 16
_CPS = _SROWS * _NIDX // _PCHUNK
_NSUP = _NCH // _CPS
_PRB = _PPW // _NIDX

_TCBLK = 2048
_RLBLK = 64


def _tc_body(x_ref, s_ref, o_ref):
    y = x_ref[...].astype(jnp.float32)
    p = jnp.dot(y, s_ref[...], preferred_element_type=jnp.float32)
    o_ref[...] = p.astype(jnp.int32)


_tc_pidx = pl.pallas_call(
    _tc_body,
    grid=(_B // _TCBLK,),
    in_specs=[
        pl.BlockSpec((_TCBLK, _L), lambda i: (i, 0)),
        pl.BlockSpec((_L, _PPR), lambda i: (0, 0)),
    ],
    out_specs=pl.BlockSpec((_TCBLK, _PPR), lambda i: (i, 0)),
    out_shape=jax.ShapeDtypeStruct((_B, _PPR), jnp.int32),
)


def _rl_body(p_ref, o_ref):
    v = p_ref[...]
    a = v[:, :_H]
    b = v[:, _H:]
    y = jnp.stack([a, b], axis=1)
    o_ref[...] = y.reshape(_RLBLK, _L, _H)


_tc_relayout = pl.pallas_call(
    _rl_body,
    grid=(_B // _RLBLK,),
    in_specs=[pl.BlockSpec((_RLBLK * _PPR, 2 * _H), lambda i: (i, 0))],
    out_specs=pl.BlockSpec((_RLBLK, _L, _H), lambda i: (i, 0, 0)),
    out_shape=jax.ShapeDtypeStruct((_B, _L, _H), jnp.float32),
)


def _build_sc():
    mesh = plsc.VectorSubcoreMesh(core_axis_name="c", subcore_axis_name="s")

    @functools.partial(
        pl.kernel,
        mesh=mesh,
        out_type=jax.ShapeDtypeStruct((_NP, 2 * _H), jnp.float32),
        scratch_types=[
            pltpu.VMEM_SHARED((100, 2 * _H), jnp.float32),
            pltpu.VMEM((_SROWS, _NIDX), jnp.int32),
            pltpu.VMEM((_SROWS, _NIDX), jnp.int32),
            pltpu.VMEM((_PCHUNK, 2 * _H), jnp.float32),
            pltpu.VMEM((_PCHUNK, 2 * _H), jnp.float32),
            pltpu.SemaphoreType.DMA,
            pltpu.SemaphoreType.DMA,
            pltpu.SemaphoreType.DMA,
            pltpu.SemaphoreType.DMA,
        ],
    )
    def k(tab2_hbm, pidx_hbm, out_hbm,
          tab_sh, psup0, psup1, rows0, rows1, sg0, sg1, so0, so1):
        P = (psup0, psup1)
        rows = (rows0, rows1)
        sg = (sg0, sg1)
        so = (so0, so1)
        cid = lax.axis_index("c")
        sid = lax.axis_index("s")
        wid = sid * 2 + cid
        pair_base = wid * _PPW
        prow_base = wid * _PRB

        @pl.when(sid == 0)
        def _stage_table():
            pltpu.sync_copy(tab2_hbm, tab_sh)

        plsc.subcore_barrier()

        def load_sup(t, pb):
            pltpu.sync_copy(
                pidx_hbm.at[pl.ds(prow_base + t * _SROWS, _SROWS)], P[pb]
            )

        def start_gather(b, pb, lrow):
            for j in range(2):
                pltpu.async_copy(
                    tab_sh.at[P[pb].at[lrow + j]],
                    rows[b].at[pl.ds(j * _NIDX, _NIDX)],
                    sg[b],
                )

        def wait_rows(b, sem):
            pltpu.make_async_copy(
                out_hbm.at[pl.ds(0, _PCHUNK)], rows[b], sem
            ).wait()

        def start_write(c, b):
            off = pair_base + c * _PCHUNK
            pltpu.async_copy(rows[b], out_hbm.at[pl.ds(off, _PCHUNK)], so[b])

        def step(c, m, nxt):
            b = m % 2
            wait_rows(b, sg[b])
            start_write(c, b)
            if nxt is not None:
                wait_rows(b, so[b])
                start_gather(b, nxt[0], nxt[1])

        load_sup(0, 0)
        start_gather(0, 0, 0)
        start_gather(1, 0, 2)

        def body(s2, carry):
            for half in (0, 1):
                t = 2 * s2 + half
                me = half
                load_sup(t + 1, 1 - me)
                for m in range(_CPS):
                    c = t * _CPS + m
                    if m < _CPS - 2:
                        nxt = (me, 2 * (m + 2))
                    else:
                        nxt = (1 - me, 2 * (m - (_CPS - 2)))
                    step(c, m, nxt)
            return carry

        lax.fori_loop(0, _NSUP // 2 - 1, body, 0)

        t = _NSUP - 2
        load_sup(t + 1, 1)
        for m in range(_CPS):
            c = t * _CPS + m
            if m < _CPS - 2:
                nxt = (0, 2 * (m + 2))
            else:
                nxt = (1, 2 * (m - (_CPS - 2)))
            step(c, m, nxt)
        t = _NSUP - 1
        for m in range(_CPS):
            c = t * _CPS + m
            nxt = (1, 2 * (m + 2)) if m < _CPS - 2 else None
            step(c, m, nxt)

        for b in (0, 1):
            wait_rows(b, so[b])

    return k


_sc_gather = _build_sc()


def _sel_matrix():
    s = np.zeros((_L, _PPR), np.float32)
    q = np.arange(_PPR)
    s[2 * q, q] = 10.0
    s[2 * q + 1, q] = 1.0
    return s


_S = _sel_matrix()


def kernel(x, table):
    tab2 = jnp.concatenate(
        [jnp.repeat(table, 10, axis=0), jnp.tile(table, (10, 1))], axis=1
    )
    pidx = _tc_pidx(x.astype(jnp.int32), jnp.asarray(_S))
    out = _sc_gather(tab2, pidx)
    return _tc_relayout(out)


# speedup vs baseline: 1.4018x; 1.4018x over previous
import functools

import jax
import jax.numpy as jnp
import numpy as np
from jax import lax
from jax.experimental import pallas as pl
from jax.experimental.pallas import tpu as pltpu
from jax.experimental.pallas import tpu_sc as plsc

_B = 16384
_L = 200
_H = 64
_BT = _B * _L
_NP = _BT // 2
_PPR = _L // 2
_NW = 32
_PPW = _NP // _NW
_PCHUNK = 200
_NIDX = 100
_NCH = _PPW // _PCHUNK
_SROWS = 16
_CPS = _SROWS * _NIDX // _PCHUNK
_NSUP = _NCH // _CPS
_PRB = _PPW // _NIDX

_TCBLK = 2048


def _tc_body(x_ref, s_ref, o_ref):
    y = x_ref[...].astype(jnp.float32)
    p = jnp.dot(y, s_ref[...], preferred_element_type=jnp.float32)
    o_ref[...] = p.astype(jnp.int32)


_tc_pidx = pl.pallas_call(
    _tc_body,
    grid=(_B // _TCBLK,),
    in_specs=[
        pl.BlockSpec((_TCBLK, _L), lambda i: (i, 0)),
        pl.BlockSpec((_L, _PPR), lambda i: (0, 0)),
    ],
    out_specs=pl.BlockSpec((_TCBLK, _PPR), lambda i: (i, 0)),
    out_shape=jax.ShapeDtypeStruct((_B, _PPR), jnp.int32),
)


def _build_sc():
    mesh = plsc.VectorSubcoreMesh(core_axis_name="c", subcore_axis_name="s")

    @functools.partial(
        pl.kernel,
        mesh=mesh,
        out_type=jax.ShapeDtypeStruct((_NP, 2 * _H), jnp.float32),
        scratch_types=[
            pltpu.VMEM_SHARED((100, 2 * _H), jnp.float32),
            pltpu.VMEM((_SROWS, _NIDX), jnp.int32),
            pltpu.VMEM((_SROWS, _NIDX), jnp.int32),
            pltpu.VMEM((_PCHUNK, 2 * _H), jnp.float32),
            pltpu.VMEM((_PCHUNK, 2 * _H), jnp.float32),
            pltpu.SemaphoreType.DMA,
            pltpu.SemaphoreType.DMA,
            pltpu.SemaphoreType.DMA,
            pltpu.SemaphoreType.DMA,
        ],
    )
    def k(tab2_hbm, pidx_hbm, out_hbm,
          tab_sh, psup0, psup1, rows0, rows1, sg0, sg1, so0, so1):
        P = (psup0, psup1)
        rows = (rows0, rows1)
        sg = (sg0, sg1)
        so = (so0, so1)
        cid = lax.axis_index("c")
        sid = lax.axis_index("s")
        wid = sid * 2 + cid
        pair_base = wid * _PPW
        prow_base = wid * _PRB

        @pl.when(sid == 0)
        def _stage_table():
            pltpu.sync_copy(tab2_hbm, tab_sh)

        plsc.subcore_barrier()

        def load_sup(t, pb):
            pltpu.sync_copy(
                pidx_hbm.at[pl.ds(prow_base + t * _SROWS, _SROWS)], P[pb]
            )

        def start_gather(b, pb, lrow):
            for j in range(2):
                pltpu.async_copy(
                    tab_sh.at[P[pb].at[lrow + j]],
                    rows[b].at[pl.ds(j * _NIDX, _NIDX)],
                    sg[b],
                )

        def wait_rows(b, sem):
            pltpu.make_async_copy(
                out_hbm.at[pl.ds(0, _PCHUNK)], rows[b], sem
            ).wait()

        def start_write(c, b):
            off = pair_base + c * _PCHUNK
            pltpu.async_copy(rows[b], out_hbm.at[pl.ds(off, _PCHUNK)], so[b])

        def step(c, m, nxt):
            b = m % 2
            wait_rows(b, sg[b])
            start_write(c, b)
            if nxt is not None:
                wait_rows(b, so[b])
                start_gather(b, nxt[0], nxt[1])

        load_sup(0, 0)
        start_gather(0, 0, 0)
        start_gather(1, 0, 2)

        def body(s2, carry):
            for half in (0, 1):
                t = 2 * s2 + half
                me = half
                load_sup(t + 1, 1 - me)
                for m in range(_CPS):
                    c = t * _CPS + m
                    if m < _CPS - 2:
                        nxt = (me, 2 * (m + 2))
                    else:
                        nxt = (1 - me, 2 * (m - (_CPS - 2)))
                    step(c, m, nxt)
            return carry

        lax.fori_loop(0, _NSUP // 2 - 1, body, 0)

        t = _NSUP - 2
        load_sup(t + 1, 1)
        for m in range(_CPS):
            c = t * _CPS + m
            if m < _CPS - 2:
                nxt = (0, 2 * (m + 2))
            else:
                nxt = (1, 2 * (m - (_CPS - 2)))
            step(c, m, nxt)
        t = _NSUP - 1
        for m in range(_CPS):
            c = t * _CPS + m
            nxt = (1, 2 * (m + 2)) if m < _CPS - 2 else None
            step(c, m, nxt)

        for b in (0, 1):
            wait_rows(b, so[b])

    return k


_sc_gather = _build_sc()


def _sel_matrix():
    s = np.zeros((_L, _PPR), np.float32)
    q = np.arange(_PPR)
    s[2 * q, q] = 10.0
    s[2 * q + 1, q] = 1.0
    return s


_S = _sel_matrix()


def kernel(x, table):
    tab2 = jnp.concatenate(
        [jnp.repeat(table, 10, axis=0), jnp.tile(table, (10, 1))], axis=1
    )
    pidx = _tc_pidx(x.astype(jnp.int32), jnp.asarray(_S))
    out = _sc_gather(tab2, pidx)
    return out.reshape(_B, _L, _H)
